# baseline (device time: 432091 ns/iter reference)
import itertools

import jax
import jax.numpy as jnp
from jax import lax
from jax.experimental import pallas as pl
from jax.experimental.pallas import tpu as pltpu

CAST_R = 1024
SIZES = (512,) + (1024,) * 7 + (512,)
OFFS = tuple(itertools.accumulate((0,) + SIZES[:-1]))
ENDS = tuple(o + s for o, s in zip(OFFS, SIZES))
A2_ROWS = (0, 2736)
CY_PIECES = ((2736, 848, 3), (3584, 1024, 4), (4608, 856, 5))
CZ_PIECES = ((5464, 1192, 6), (6656, 1024, 7), (7680, 512, 8))
A_TRIG = tuple((ENDS[k] - 1) // CAST_R for k in range(len(SIZES)))


def kernel(x):
    m, n = x.shape
    qm = m // 4
    cpq = qm // CAST_R
    nc = m // CAST_R
    nk = len(SIZES)

    def body(x_ref, out_ref, vin, vout, in_sems, out_sems, send_sems, recv_sems):
        my_x = lax.axis_index("x")
        my_y = lax.axis_index("y")
        my_z = lax.axis_index("z")

        mine = my_x * m
        other = (1 - my_x) * m
        q_mine = 2 * my_y + my_z
        q_bz = 2 * my_y + (1 - my_z)
        q_by = 2 * (1 - my_y) + my_z
        q_diag = 2 * (1 - my_y) + (1 - my_z)

        A, A2, BZ, BY, CY, CZ = range(6)

        x_nbr = (1 - my_x, my_y, my_z)
        y_nbr = (my_x, 1 - my_y, my_z)
        z_nbr = (my_x, my_y, 1 - my_z)

        def remote(phase, k, row, sz, dev):
            return pltpu.make_async_remote_copy(
                src_ref=out_ref.at[pl.ds(row, sz)],
                dst_ref=out_ref.at[pl.ds(row, sz)],
                send_sem=send_sems.at[phase, k],
                recv_sem=recv_sems.at[phase, k],
                device_id=dev,
                device_id_type=pl.DeviceIdType.MESH,
            )

        rdmas = {}

        cast_quarters = [q_mine, q_diag, q_bz, q_by]

        def cast_row(j):
            return cast_quarters[j // cpq] * qm + (j % cpq) * CAST_R

        def in_copy(j, slot):
            return pltpu.make_async_copy(
                x_ref.at[pl.ds(cast_row(j), CAST_R)], vin.at[slot], in_sems.at[slot]
            )

        a2_trig = cpq + (A2_ROWS[1] - 1) // CAST_R

        pending_store = [None, None]
        in_copy(0, 0).start()
        for j in range(nc):
            slot = j % 2
            if j + 1 < nc:
                in_copy(j + 1, 1 - slot).start()
            in_copy(j, slot).wait()
            if pending_store[slot] is not None:
                pending_store[slot].wait()
                pending_store[slot] = None
            vout[slot] = vin[slot].astype(jnp.bfloat16)
            st = pltpu.make_async_copy(
                vout.at[slot],
                out_ref.at[pl.ds(mine + cast_row(j), CAST_R)],
                out_sems.at[slot],
            )
            st.start()
            a_ready = [k for k in range(nk) if A_TRIG[k] == j] if j < cpq else []
            if a_ready:
                st.wait()
                for k in a_ready:
                    rd = remote(A, k, mine + q_mine * qm + OFFS[k], SIZES[k], x_nbr)
                    rd.start()
                    rdmas[(A, k)] = rd
            elif j == a2_trig:
                st.wait()
                rd = remote(A2, 0, mine + q_diag * qm + A2_ROWS[0], A2_ROWS[1], x_nbr)
                rd.start()
                rdmas[(A2, 0)] = rd
            else:
                pending_store[slot] = st
        for st in pending_store:
            if st is not None:
                st.wait()

        def process_bz(k):
            rdmas[(BZ, k)].wait_recv()
            for i, (off, sz, dep) in enumerate(CY_PIECES):
                if dep == k:
                    rd = remote(CY, i, other + q_bz * qm + off, sz, y_nbr)
                    rd.start()
                    rdmas[(CY, i)] = rd

        def process_by(k):
            rdmas[(BY, k)].wait_recv()
            for i, (off, sz, dep) in enumerate(CZ_PIECES):
                if dep == k:
                    rd = remote(CZ, i, other + q_by * qm + off, sz, z_nbr)
                    rd.start()
                    rdmas[(CZ, i)] = rd

        for k in range(nk):
            rdmas[(A, k)].wait_recv()
            row = other + q_mine * qm + OFFS[k]
            rd = remote(BZ, k, row, SIZES[k], z_nbr)
            rd.start()
            rdmas[(BZ, k)] = rd
            rd = remote(BY, k, row, SIZES[k], y_nbr)
            rd.start()
            rdmas[(BY, k)] = rd
            if k >= 1:
                process_bz(k - 1)
                process_by(k - 1)
        process_bz(nk - 1)
        process_by(nk - 1)

        rdmas[(A2, 0)].wait_recv()
        for i in range(len(CY_PIECES)):
            rdmas[(CY, i)].wait_recv()
        for i in range(len(CZ_PIECES)):
            rdmas[(CZ, i)].wait_recv()
        for rd in rdmas.values():
            rd.wait_send()

    return pl.pallas_call(
        body,
        out_shape=jax.ShapeDtypeStruct((2 * m, n), jnp.bfloat16),
        in_specs=[pl.BlockSpec(memory_space=pl.ANY)],
        out_specs=pl.BlockSpec(memory_space=pl.ANY),
        scratch_shapes=[
            pltpu.VMEM((2, CAST_R, n), x.dtype),
            pltpu.VMEM((2, CAST_R, n), jnp.bfloat16),
            pltpu.SemaphoreType.DMA((2,)),
            pltpu.SemaphoreType.DMA((2,)),
            pltpu.SemaphoreType.DMA((6, 9)),
            pltpu.SemaphoreType.DMA((6, 9)),
        ],
    )(x)


# device time: 407897 ns/iter; 1.0593x vs baseline; 1.0593x over previous
import itertools

import jax
import jax.numpy as jnp
from jax import lax
from jax.experimental import pallas as pl
from jax.experimental.pallas import tpu as pltpu

CAST_R = 1024
SIZES = (512,) + (1024,) * 7 + (512,)
OFFS = tuple(itertools.accumulate((0,) + SIZES[:-1]))
ENDS = tuple(o + s for o, s in zip(OFFS, SIZES))
DIAG = ((0, 2736), (2736, 2728), (5464, 2728))
CY_DEP = min(k for k in range(len(SIZES)) if ENDS[k] >= DIAG[1][0] + DIAG[1][1])
CZ_DEP = min(k for k in range(len(SIZES)) if ENDS[k] >= DIAG[2][0] + DIAG[2][1])
A_TRIG = tuple((ENDS[k] - 1) // CAST_R for k in range(len(SIZES)))


def kernel(x):
    m, n = x.shape
    qm = m // 4
    cpq = qm // CAST_R
    nc = m // CAST_R
    nk = len(SIZES)

    def body(x_ref, out_ref, vin, vout, in_sems, out_sems, send_sems, recv_sems):
        my_x = lax.axis_index("x")
        my_y = lax.axis_index("y")
        my_z = lax.axis_index("z")

        mine = my_x * m
        other = (1 - my_x) * m
        q_mine = 2 * my_y + my_z
        q_bz = 2 * my_y + (1 - my_z)
        q_by = 2 * (1 - my_y) + my_z
        q_diag = 2 * (1 - my_y) + (1 - my_z)

        A, A2, BZ, BY, CY, CZ = range(6)

        x_nbr = (1 - my_x, my_y, my_z)
        y_nbr = (my_x, 1 - my_y, my_z)
        z_nbr = (my_x, my_y, 1 - my_z)

        def remote(phase, k, row, sz, dev):
            return pltpu.make_async_remote_copy(
                src_ref=out_ref.at[pl.ds(row, sz)],
                dst_ref=out_ref.at[pl.ds(row, sz)],
                send_sem=send_sems.at[phase, k],
                recv_sem=recv_sems.at[phase, k],
                device_id=dev,
                device_id_type=pl.DeviceIdType.MESH,
            )

        rdmas = {}

        cast_quarters = [q_mine, q_diag, q_bz, q_by]

        def cast_row(j):
            return cast_quarters[j // cpq] * qm + (j % cpq) * CAST_R

        def in_copy(j, slot):
            return pltpu.make_async_copy(
                x_ref.at[pl.ds(cast_row(j), CAST_R)], vin.at[slot], in_sems.at[slot]
            )

        a2_trig = cpq + (DIAG[0][1] - 1) // CAST_R

        pending_store = [None, None]
        in_copy(0, 0).start()
        for j in range(nc):
            slot = j % 2
            if j + 1 < nc:
                in_copy(j + 1, 1 - slot).start()
            in_copy(j, slot).wait()
            if pending_store[slot] is not None:
                pending_store[slot].wait()
                pending_store[slot] = None
            vout[slot] = vin[slot].astype(jnp.bfloat16)
            st = pltpu.make_async_copy(
                vout.at[slot],
                out_ref.at[pl.ds(mine + cast_row(j), CAST_R)],
                out_sems.at[slot],
            )
            st.start()
            a_ready = [k for k in range(nk) if A_TRIG[k] == j] if j < cpq else []
            if a_ready:
                st.wait()
                for k in a_ready:
                    rd = remote(A, k, mine + q_mine * qm + OFFS[k], SIZES[k], x_nbr)
                    rd.start()
                    rdmas[(A, k)] = rd
            elif j == a2_trig:
                st.wait()
                rd = remote(A2, 0, mine + q_diag * qm + DIAG[0][0], DIAG[0][1], x_nbr)
                rd.start()
                rdmas[(A2, 0)] = rd
            else:
                pending_store[slot] = st
        for st in pending_store:
            if st is not None:
                st.wait()

        for k in range(nk):
            rdmas[(A, k)].wait_recv()
            row = other + q_mine * qm + OFFS[k]
            rd = remote(BZ, k, row, SIZES[k], z_nbr)
            rd.start()
            rdmas[(BZ, k)] = rd
            rd = remote(BY, k, row, SIZES[k], y_nbr)
            rd.start()
            rdmas[(BY, k)] = rd

        for k in range(nk):
            rdmas[(BZ, k)].wait_recv()
            if k == CY_DEP:
                rd = remote(CY, 0, other + q_bz * qm + DIAG[1][0], DIAG[1][1], y_nbr)
                rd.start()
                rdmas[(CY, 0)] = rd
        for k in range(nk):
            rdmas[(BY, k)].wait_recv()
            if k == CZ_DEP:
                rd = remote(CZ, 0, other + q_by * qm + DIAG[2][0], DIAG[2][1], z_nbr)
                rd.start()
                rdmas[(CZ, 0)] = rd

        for phase in (A2, CY, CZ):
            rdmas[(phase, 0)].wait_recv()
        for rd in rdmas.values():
            rd.wait_send()

    return pl.pallas_call(
        body,
        out_shape=jax.ShapeDtypeStruct((2 * m, n), jnp.bfloat16),
        in_specs=[pl.BlockSpec(memory_space=pl.ANY)],
        out_specs=pl.BlockSpec(memory_space=pl.ANY),
        scratch_shapes=[
            pltpu.VMEM((2, CAST_R, n), x.dtype),
            pltpu.VMEM((2, CAST_R, n), jnp.bfloat16),
            pltpu.SemaphoreType.DMA((2,)),
            pltpu.SemaphoreType.DMA((2,)),
            pltpu.SemaphoreType.DMA((6, 9)),
            pltpu.SemaphoreType.DMA((6, 9)),
        ],
    )(x)
